# baseline (speedup 1.0000x reference)
import jax
import jax.numpy as jnp
from jax.experimental import pallas as pl
from jax.experimental.pallas import tpu as pltpu

N_NODES_ = 10000
IN_F_ = 256
HID_F_ = 64
OUT_F_ = 64
N_HEADS_ = 4
GRID_ = 5


def _gal_two_rows(x2, W, b, a_col, n00, n01, n10, n11):
    out_f = W.shape[0]
    xw = jax.lax.dot_general(
        x2, W, (((1,), (1,)), ((), ())),
        preferred_element_type=jnp.float32) + b
    x0 = xw[0:1, :]
    x1 = xw[1:2, :]
    u = jax.lax.dot_general(
        xw, a_col[0:out_f, :], (((1,), (0,)), ((), ())),
        preferred_element_type=jnp.float32)
    v = jax.lax.dot_general(
        xw, a_col[out_f:2 * out_f, :], (((1,), (0,)), ((), ())),
        preferred_element_type=jnp.float32)
    u0, u1, v0, v1 = u[0, 0], u[1, 0], v[0, 0], v[1, 0]

    def lrelu(z):
        return jnp.where(z >= 0, z, 0.01 * z)

    e00 = lrelu(u0 + v0)
    e01 = lrelu(u0 + v1)
    e10 = lrelu(u1 + v0)
    e11 = lrelu(u1 + v1)

    def cmul(n, val):
        return jnp.where(n == 0, jnp.float32(0.0), n * val)

    eall0 = cmul(n00, e00) + cmul(n10, e10)
    eall1 = cmul(n01, e01) + cmul(n11, e11)
    w00 = jnp.exp(e00 - eall0)
    w10 = jnp.exp(e10 - eall0)
    w01 = jnp.exp(e01 - eall1)
    w11 = jnp.exp(e11 - eall1)
    out0 = cmul(n00, w00) * x0 + cmul(n10, w10) * x1
    out1 = cmul(n01, w01) * x0 + cmul(n11, w11) * x1
    return jnp.concatenate([out0, out1], axis=0)


def _gat_body(ei_ref, x_ref, wh_ref, bh_ref, ah_ref, wo_ref, bo_ref, ao_ref,
              out_ref, acc_ref):
    f32 = jnp.float32
    i = pl.program_id(0)

    @pl.when(i == 0)
    def _init():
        acc_ref[0] = f32(0.0)
        acc_ref[1] = f32(0.0)
        acc_ref[2] = f32(0.0)

    r = (ei_ref[0:1, :] > 0).astype(f32)
    c = (ei_ref[1:2, :] > 0).astype(f32)
    acc_ref[0] += jnp.sum(r)
    acc_ref[1] += jnp.sum(c)
    acc_ref[2] += jnp.sum(r * c)

    const = -jnp.log(f32(OUT_F_))
    out_ref[...] = jnp.full(out_ref.shape, const, dtype=f32)

    @pl.when(i == GRID_ - 1)
    def _final():
        s_r = acc_ref[0]
        s_c = acc_ref[1]
        s_rc = acc_ref[2]
        n_edges = f32(ei_ref.shape[1] * GRID_)
        n11 = s_rc
        n10 = s_r - s_rc
        n01 = s_c - s_rc
        n00 = n_edges - s_r - s_c + s_rc

        x2 = x_ref[0:2, :]
        heads = []
        for h in range(N_HEADS_):
            heads.append(_gal_two_rows(
                x2, wh_ref[h], bh_ref[h:h + 1, :], ah_ref[h],
                n00, n01, n10, n11))
        h2 = jnp.concatenate(heads, axis=1)

        o2 = _gal_two_rows(
            h2, wo_ref[...], bo_ref[...], ao_ref[...],
            n00, n01, n10, n11)

        oe = jnp.where(o2 > 0, o2, jnp.exp(jnp.minimum(o2, 0.0)) - 1.0)
        m = jnp.max(oe, axis=1, keepdims=True)
        s = oe - m
        ls = s - jnp.log(jnp.sum(jnp.exp(s), axis=1, keepdims=True))
        out_ref[0:2, :] = ls


def kernel(x, edge_index, W_h, b_h, a_h, W_o, b_o, a_o):
    bo = b_o.reshape(1, OUT_F_)
    n_edges = edge_index.shape[1]
    chunk = n_edges // GRID_
    rows_per_block = N_NODES_ // GRID_
    out = pl.pallas_call(
        _gat_body,
        out_shape=jax.ShapeDtypeStruct((N_NODES_, OUT_F_), jnp.float32),
        grid=(GRID_,),
        in_specs=[
            pl.BlockSpec((2, chunk), lambda i: (0, i)),
            pl.BlockSpec((8, IN_F_), lambda i: (0, 0)),
            pl.BlockSpec(W_h.shape, lambda i: (0, 0, 0)),
            pl.BlockSpec(b_h.shape, lambda i: (0, 0)),
            pl.BlockSpec(a_h.shape, lambda i: (0, 0, 0)),
            pl.BlockSpec(W_o.shape, lambda i: (0, 0)),
            pl.BlockSpec((1, OUT_F_), lambda i: (0, 0)),
            pl.BlockSpec(a_o.shape, lambda i: (0, 0)),
        ],
        out_specs=pl.BlockSpec(
            (rows_per_block, OUT_F_), lambda i: ((i + 1) % GRID_, 0)),
        scratch_shapes=[pltpu.SMEM((4,), jnp.float32)],
    )(edge_index, x, W_h, b_h, a_h, W_o, bo, a_o)
    return out


# speedup vs baseline: 1.0812x; 1.0812x over previous
import jax
import jax.numpy as jnp
from jax.experimental import pallas as pl
from jax.experimental.pallas import tpu as pltpu

N_NODES_ = 10000
IN_F_ = 256
HID_F_ = 64
OUT_F_ = 64
N_HEADS_ = 4
GRID_ = 2


def _gal_two_rows(x2, W, b, a_col, n00, n01, n10, n11):
    out_f = W.shape[0]
    xw = jax.lax.dot_general(
        x2, W, (((1,), (1,)), ((), ())),
        preferred_element_type=jnp.float32) + b
    x0 = xw[0:1, :]
    x1 = xw[1:2, :]
    u = jax.lax.dot_general(
        xw, a_col[0:out_f, :], (((1,), (0,)), ((), ())),
        preferred_element_type=jnp.float32)
    v = jax.lax.dot_general(
        xw, a_col[out_f:2 * out_f, :], (((1,), (0,)), ((), ())),
        preferred_element_type=jnp.float32)
    u0, u1, v0, v1 = u[0, 0], u[1, 0], v[0, 0], v[1, 0]

    def lrelu(z):
        return jnp.where(z >= 0, z, 0.01 * z)

    e00 = lrelu(u0 + v0)
    e01 = lrelu(u0 + v1)
    e10 = lrelu(u1 + v0)
    e11 = lrelu(u1 + v1)

    def cmul(n, val):
        return jnp.where(n == 0, jnp.float32(0.0), n * val)

    eall0 = cmul(n00, e00) + cmul(n10, e10)
    eall1 = cmul(n01, e01) + cmul(n11, e11)
    w00 = jnp.exp(e00 - eall0)
    w10 = jnp.exp(e10 - eall0)
    w01 = jnp.exp(e01 - eall1)
    w11 = jnp.exp(e11 - eall1)
    out0 = cmul(n00, w00) * x0 + cmul(n10, w10) * x1
    out1 = cmul(n01, w01) * x0 + cmul(n11, w11) * x1
    return jnp.concatenate([out0, out1], axis=0)


def _gat_body(ei_ref, x_ref, wh_ref, bh_ref, ah_ref, wo_ref, bo_ref, ao_ref,
              out_ref, acc_ref):
    f32 = jnp.float32
    i = pl.program_id(0)

    @pl.when(i == 0)
    def _init():
        acc_ref[0] = f32(0.0)
        acc_ref[1] = f32(0.0)
        acc_ref[2] = f32(0.0)

    r = (ei_ref[0:1, :] > 0).astype(f32)
    c = (ei_ref[1:2, :] > 0).astype(f32)
    acc_ref[0] += jnp.sum(r)
    acc_ref[1] += jnp.sum(c)
    acc_ref[2] += jnp.sum(r * c)

    const = -jnp.log(f32(OUT_F_))
    out_ref[...] = jnp.full(out_ref.shape, const, dtype=f32)

    @pl.when(i == GRID_ - 1)
    def _final():
        s_r = acc_ref[0]
        s_c = acc_ref[1]
        s_rc = acc_ref[2]
        n_edges = f32(ei_ref.shape[1] * GRID_)
        n11 = s_rc
        n10 = s_r - s_rc
        n01 = s_c - s_rc
        n00 = n_edges - s_r - s_c + s_rc

        x2 = x_ref[0:2, :]
        heads = []
        for h in range(N_HEADS_):
            heads.append(_gal_two_rows(
                x2, wh_ref[h], bh_ref[h:h + 1, :], ah_ref[h],
                n00, n01, n10, n11))
        h2 = jnp.concatenate(heads, axis=1)

        o2 = _gal_two_rows(
            h2, wo_ref[...], bo_ref[...], ao_ref[...],
            n00, n01, n10, n11)

        oe = jnp.where(o2 > 0, o2, jnp.exp(jnp.minimum(o2, 0.0)) - 1.0)
        m = jnp.max(oe, axis=1, keepdims=True)
        s = oe - m
        ls = s - jnp.log(jnp.sum(jnp.exp(s), axis=1, keepdims=True))
        out_ref[0:2, :] = ls


def kernel(x, edge_index, W_h, b_h, a_h, W_o, b_o, a_o):
    bo = b_o.reshape(1, OUT_F_)
    n_edges = edge_index.shape[1]
    chunk = n_edges // GRID_
    rows_per_block = N_NODES_ // GRID_
    out = pl.pallas_call(
        _gat_body,
        out_shape=jax.ShapeDtypeStruct((N_NODES_, OUT_F_), jnp.float32),
        grid=(GRID_,),
        in_specs=[
            pl.BlockSpec((2, chunk), lambda i: (0, i)),
            pl.BlockSpec((8, IN_F_), lambda i: (0, 0)),
            pl.BlockSpec(W_h.shape, lambda i: (0, 0, 0)),
            pl.BlockSpec(b_h.shape, lambda i: (0, 0)),
            pl.BlockSpec(a_h.shape, lambda i: (0, 0, 0)),
            pl.BlockSpec(W_o.shape, lambda i: (0, 0)),
            pl.BlockSpec((1, OUT_F_), lambda i: (0, 0)),
            pl.BlockSpec(a_o.shape, lambda i: (0, 0)),
        ],
        out_specs=pl.BlockSpec(
            (rows_per_block, OUT_F_), lambda i: ((i + 1) % GRID_, 0)),
        scratch_shapes=[pltpu.SMEM((4,), jnp.float32)],
    )(edge_index, x, W_h, b_h, a_h, W_o, bo, a_o)
    return out
